# baseline (speedup 1.0000x reference)
import functools

import jax
import jax.numpy as jnp
from jax import lax
from jax.experimental import pallas as pl
from jax.experimental.pallas import tpu as pltpu
from jax.experimental.pallas import tpu_sc as plsc

DIM = 64
L = 16
NC, NS = 2, 16
NW = NC * NS
BLK = 128
NV = 1000000
NBLK = -(-NV // BLK)
BPW_BLK = -(-NBLK // NW)
SLIVER0 = (NBLK - 1) * BLK
SLIVER_W = NV - SLIVER0
BATCH = 16384
TRASH = BATCH
STAG_ROWS = BATCH + 8
NBUF = 4
POSB = 14


def _dyn_lane(v, i):
    g = v.at[jnp.full((L,), i, jnp.int32)].get(mode="promise_in_bounds")
    return g[0]


def _p1_body(oid_hbm, eid_hbm, mt_hbm, me_hbm, ost_hbm, est_hbm,
             allidx_v, bkt_v, wl_v, chunk_v, sliver_v, ring_v, posring_v,
             cnt_s, sem_c, sem_s):
    wid = lax.axis_index("s") * NC + lax.axis_index("c")
    rlo = wid * (BPW_BLK * BLK)
    rhi = rlo + BPW_BLK * BLK
    lanei = lax.iota(jnp.int32, L)
    trash = jnp.full((L,), TRASH, jnp.int32)

    def one_pass(tbl, idxh, stag):
        for t in range(128 // L):
            posring_v[pl.ds(t * L, L)] = trash
        cnt_s[0] = jnp.int32(0)
        pltpu.sync_copy(idxh, allidx_v)

        def bk(v, nb):
            row = v >> 3
            seg = v & 7
            iv = allidx_v[row, pl.ds(seg * L, L)]
            m = (iv >= rlo) & (iv < rhi)
            packed = ((iv - rlo) << POSB) | (v * L + lanei)
            plsc.store_compressed(bkt_v.at[pl.ds(nb, L)], packed, mask=m)
            return nb + plsc.all_reduce_population_count(m)[0]

        nb = lax.fori_loop(0, BATCH // L, bk, jnp.int32(0))
        ngv = (nb + L - 1) >> 4
        sliver_loc = jnp.int32(SLIVER0) - rlo

        def extract_entry(ic_s, pos_s, buf):
            slot = cnt_s[0]
            colv = jnp.full((L,), ic_s, jnp.int32)
            for k in range(DIM // L):
                seg = plsc.load_gather(buf, [lanei + k * L, colv])
                ring_v[slot, pl.ds(k * L, L)] = seg
            base = (slot >> 4) * L
            pr = posring_v[pl.ds(base, L)]
            posring_v[pl.ds(base, L)] = jnp.where(lanei == (slot & (L - 1)),
                                                  pos_s, pr)
            nslot = slot + 1

            @pl.when(nslot == 128)
            def _():
                pltpu.async_copy(ring_v, stag.at[posring_v], sem_s).wait()
                for t in range(128 // L):
                    posring_v[pl.ds(t * L, L)] = trash

            cnt_s[0] = jnp.where(nslot == 128, 0, nslot)

        def process(buf, match, colbase):
            def pg(g, w):
                wv = bkt_v[pl.ds(g * L, L)]
                il = lax.shift_right_logical(wv, POSB)
                valid = (lanei + g * L) < nb
                m = match(il) & valid
                plsc.store_compressed(wl_v.at[pl.ds(w, L)], wv, mask=m)
                return w + plsc.all_reduce_population_count(m)[0]

            w = lax.fori_loop(0, ngv, pg, jnp.int32(0))

            def pe(e, _):
                grp = wl_v[pl.ds((e >> 4) * L, L)]
                sc = _dyn_lane(grp, e & (L - 1))
                il_s = lax.shift_right_logical(sc, POSB)
                pos_s = sc & ((1 << POSB) - 1)
                extract_entry(il_s - colbase, pos_s, buf)
                return 0

            lax.fori_loop(0, w, pe, 0)

        def fire(t):
            @pl.when(t < BPW_BLK)
            def _():
                blk = jnp.minimum(wid * BPW_BLK + t, NBLK - 2)
                start = pl.multiple_of(blk * BLK, BLK)
                for jb in range(DIM // 8):
                    pltpu.async_copy(
                        tbl.at[pl.ds(jb * 8, 8), pl.ds(start, BLK)],
                        chunk_v.at[t & (NBUF - 1), pl.ds(jb * 8, 8), :],
                        sem_c)

            @pl.when(t == BPW_BLK)
            def _():
                pltpu.async_copy(tbl.at[:, pl.ds(SLIVER0, SLIVER_W)],
                                 sliver_v, sem_c)

        for t in range(NBUF - 1):
            fire(jnp.int32(t))

        def chunk_step(s, _):
            fire(s + (NBUF - 1))
            pltpu.make_async_copy(tbl.at[:, pl.ds(0, BLK)],
                                  chunk_v.at[s & (NBUF - 1)], sem_c).wait()
            process(chunk_v.at[s & (NBUF - 1)],
                    lambda il: ((il >> 7) == s) & (il < sliver_loc),
                    s * BLK)
            return 0

        lax.fori_loop(0, BPW_BLK, chunk_step, 0)
        pltpu.make_async_copy(tbl.at[:, pl.ds(SLIVER0, SLIVER_W)],
                              sliver_v, sem_c).wait()
        process(sliver_v, lambda il: il >= sliver_loc, sliver_loc)
        pltpu.async_copy(ring_v, stag.at[posring_v], sem_s).wait()

    one_pass(mt_hbm, oid_hbm, ost_hbm)
    one_pass(me_hbm, eid_hbm, est_hbm)


def _p2_body(ost_hbm, est_hbm, out_hbm, obuf_v, ebuf_v, out_v, sem):
    wid = lax.axis_index("s") * NC + lax.axis_index("c")
    bpw = BATCH // NW
    base = wid * bpw
    lane = lax.iota(jnp.int32, L)

    def fire(c):
        pltpu.async_copy(ost_hbm.at[pl.ds(base + c * 128, 128), :],
                         obuf_v.at[c & 1], sem)
        pltpu.async_copy(est_hbm.at[pl.ds(base + c * 128, 128), :],
                         ebuf_v.at[c & 1], sem)

    fire(jnp.int32(0))

    def chunk(c, _):
        @pl.when(c < bpw // 128 - 1)
        def _():
            fire(c + 1)

        pltpu.make_async_copy(ost_hbm.at[pl.ds(0, 128), :],
                              obuf_v.at[c & 1], sem).wait()
        pltpu.make_async_copy(est_hbm.at[pl.ds(0, 128), :],
                              ebuf_v.at[c & 1], sem).wait()

        def group(j, _):
            dotv = jnp.zeros((L,), jnp.float32)
            pv = jnp.zeros((L,), jnp.float32)
            for r in range(L):
                row = j * L + r
                dot = jnp.zeros((L,), jnp.float32)
                on = jnp.zeros((L,), jnp.float32)
                en = jnp.zeros((L,), jnp.float32)
                for k in range(DIM // L):
                    o = obuf_v[c & 1, row, pl.ds(k * L, L)]
                    e = ebuf_v[c & 1, row, pl.ds(k * L, L)]
                    dot = dot + o * e
                    on = on + o * o
                    en = en + e * e
                sdot = jnp.sum(dot)
                sp = jnp.sum(on) * jnp.sum(en)
                dotv = jnp.where(lane == r, sdot, dotv)
                pv = jnp.where(lane == r, sp, pv)
            yi = jnp.int32(0x5F3759DF) - lax.shift_right_logical(
                plsc.bitcast(pv, jnp.int32), 1)
            y = plsc.bitcast(yi, jnp.float32)
            xh = pv * jnp.float32(0.5)
            for _ in range(3):
                y = y * (jnp.float32(1.5) - xh * y * y)
            out_v[pl.ds(c * 128 + j * L, L)] = dotv * y
            return 0

        lax.fori_loop(0, 128 // L, group, 0)
        return 0

    lax.fori_loop(0, bpw // 128, chunk, 0)
    pltpu.sync_copy(out_v, out_hbm.at[pl.ds(base, bpw)])


def kernel(o_id, e_id, mentors, mentees):
    batch = o_id.shape[0]
    mt = mentors.T
    me = mentees.T
    oid2 = o_id.reshape(batch // 128, 128)
    eid2 = e_id.reshape(batch // 128, 128)

    mesh = plsc.VectorSubcoreMesh(core_axis_name="c", subcore_axis_name="s",
                                  num_cores=NC, num_subcores=NS)
    params = pltpu.CompilerParams(needs_layout_passes=False,
                                  use_tc_tiling_on_sc=True)
    p1 = pl.kernel(
        _p1_body,
        out_type=(jax.ShapeDtypeStruct((STAG_ROWS, 2 * DIM), jnp.float32),
                  jax.ShapeDtypeStruct((STAG_ROWS, 2 * DIM), jnp.float32)),
        mesh=mesh,
        compiler_params=params,
        scratch_types=[
            pltpu.VMEM((batch // 128, 128), jnp.int32),
            pltpu.VMEM((batch + L,), jnp.int32),
            pltpu.VMEM((batch + L,), jnp.int32),
            pltpu.VMEM((NBUF, DIM, BLK), jnp.float32),
            pltpu.VMEM((DIM, SLIVER_W), jnp.float32),
            pltpu.VMEM((128, 2 * DIM), jnp.float32),
            pltpu.VMEM((128,), jnp.int32),
            pltpu.SMEM((4,), jnp.int32),
            pltpu.SemaphoreType.DMA,
            pltpu.SemaphoreType.DMA,
        ],
    )
    ost, est = p1(oid2, eid2, mt, me)

    p2 = pl.kernel(
        _p2_body,
        out_type=jax.ShapeDtypeStruct((batch,), jnp.float32),
        mesh=mesh,
        compiler_params=params,
        scratch_types=[
            pltpu.VMEM((2, 128, 2 * DIM), jnp.float32),
            pltpu.VMEM((2, 128, 2 * DIM), jnp.float32),
            pltpu.VMEM((batch // NW,), jnp.float32),
            pltpu.SemaphoreType.DMA,
        ],
    )
    return p2(ost, est)


# speedup vs baseline: 1.0068x; 1.0068x over previous
import functools

import jax
import jax.numpy as jnp
from jax import lax
from jax.experimental import pallas as pl
from jax.experimental.pallas import tpu as pltpu
from jax.experimental.pallas import tpu_sc as plsc

DIM = 64
L = 16
NC, NS = 2, 16
NW = NC * NS
BLK = 128
NV = 1000000
NBLK = -(-NV // BLK)
BPW_BLK = -(-NBLK // NW)
WINB = 2
WINW = WINB * BLK
NWIN = -(-BPW_BLK // WINB)
BMAX = NBLK - 3
SLIVER0 = (NBLK - 2) * BLK
SLIVER_W = NV - SLIVER0
BATCH = 16384
TRASH = BATCH
STAG_ROWS = BATCH + 8
NBUF = 2
POSB = 14


def _dyn_lane(v, i):
    g = v.at[jnp.full((L,), i, jnp.int32)].get(mode="promise_in_bounds")
    return g[0]


def _p1_body(oid_hbm, eid_hbm, mt_hbm, me_hbm, ost_hbm, est_hbm,
             allidx_v, bkt_v, wl_v, chunk_v, sliver_v, ring_v, posring_v,
             cnt_s, sem_c, sem_s):
    wid = lax.axis_index("s") * NC + lax.axis_index("c")
    rlo = wid * (BPW_BLK * BLK)
    rhi = rlo + BPW_BLK * BLK
    lanei = lax.iota(jnp.int32, L)
    trash = jnp.full((L,), TRASH, jnp.int32)

    def one_pass(tbl, idxh, stag):
        for t in range(128 // L):
            posring_v[pl.ds(t * L, L)] = trash
        cnt_s[0] = jnp.int32(0)
        pltpu.sync_copy(idxh, allidx_v)

        def bk(v, nb):
            row = v >> 3
            seg = v & 7
            iv = allidx_v[row, pl.ds(seg * L, L)]
            m = (iv >= rlo) & (iv < rhi)
            packed = ((iv - rlo) << POSB) | (v * L + lanei)
            plsc.store_compressed(bkt_v.at[pl.ds(nb, L)], packed, mask=m)
            return nb + plsc.all_reduce_population_count(m)[0]

        nb = lax.fori_loop(0, BATCH // L, bk, jnp.int32(0))
        ngv = (nb + L - 1) >> 4
        sliver_loc = jnp.int32(SLIVER0) - rlo

        def extract_entry(ic_s, pos_s, buf):
            slot = cnt_s[0]
            colv = jnp.full((L,), ic_s, jnp.int32)
            for k in range(DIM // L):
                seg = plsc.load_gather(buf, [lanei + k * L, colv])
                ring_v[slot, pl.ds(k * L, L)] = seg
            base = (slot >> 4) * L
            pr = posring_v[pl.ds(base, L)]
            posring_v[pl.ds(base, L)] = jnp.where(lanei == (slot & (L - 1)),
                                                  pos_s, pr)
            nslot = slot + 1

            @pl.when(nslot == 128)
            def _():
                pltpu.async_copy(ring_v, stag.at[posring_v], sem_s).wait()
                for t in range(128 // L):
                    posring_v[pl.ds(t * L, L)] = trash

            cnt_s[0] = jnp.where(nslot == 128, 0, nslot)

        def process(buf, match, colbase):
            def pg(g, w):
                wv = bkt_v[pl.ds(g * L, L)]
                il = lax.shift_right_logical(wv, POSB)
                valid = (lanei + g * L) < nb
                m = match(il) & valid
                plsc.store_compressed(wl_v.at[pl.ds(w, L)], wv, mask=m)
                return w + plsc.all_reduce_population_count(m)[0]

            w = lax.fori_loop(0, ngv, pg, jnp.int32(0))

            def pe(e, _):
                grp = wl_v[pl.ds((e >> 4) * L, L)]
                sc = _dyn_lane(grp, e & (L - 1))
                il_s = lax.shift_right_logical(sc, POSB)
                pos_s = sc & ((1 << POSB) - 1)
                extract_entry(il_s - colbase, pos_s, buf)
                return 0

            lax.fori_loop(0, w, pe, 0)

        def fire(t):
            @pl.when(t < NWIN)
            def _():
                blk = jnp.minimum(wid * BPW_BLK + t * WINB, BMAX)
                start = pl.multiple_of(blk * BLK, BLK)
                for jb in range(DIM // 8):
                    pltpu.async_copy(
                        tbl.at[pl.ds(jb * 8, 8), pl.ds(start, WINW)],
                        chunk_v.at[t & (NBUF - 1), pl.ds(jb * 8, 8), :],
                        sem_c)

            @pl.when(t == NWIN)
            def _():
                for jb in range(DIM // 8):
                    pltpu.async_copy(
                        tbl.at[pl.ds(jb * 8, 8), pl.ds(SLIVER0, SLIVER_W)],
                        sliver_v.at[pl.ds(jb * 8, 8), :], sem_c)

        for t in range(NBUF - 1):
            fire(jnp.int32(t))

        def chunk_step(s, _):
            fire(s + (NBUF - 1))
            pltpu.make_async_copy(tbl.at[:, pl.ds(0, WINW)],
                                  chunk_v.at[s & (NBUF - 1)], sem_c).wait()
            process(chunk_v.at[s & (NBUF - 1)],
                    lambda il: ((il >> 8) == s) & (il < sliver_loc),
                    s * WINW)
            return 0

        lax.fori_loop(0, NWIN, chunk_step, 0)
        pltpu.make_async_copy(tbl.at[:, pl.ds(SLIVER0, SLIVER_W)],
                              sliver_v, sem_c).wait()
        process(sliver_v, lambda il: il >= sliver_loc, sliver_loc)
        pltpu.async_copy(ring_v, stag.at[posring_v], sem_s).wait()

    one_pass(mt_hbm, oid_hbm, ost_hbm)
    one_pass(me_hbm, eid_hbm, est_hbm)


def _p2_body(ost_hbm, est_hbm, out_hbm, obuf_v, ebuf_v, out_v, sem):
    wid = lax.axis_index("s") * NC + lax.axis_index("c")
    bpw = BATCH // NW
    base = wid * bpw
    lane = lax.iota(jnp.int32, L)

    def fire(c):
        pltpu.async_copy(ost_hbm.at[pl.ds(base + c * 128, 128), :],
                         obuf_v.at[c & 1], sem)
        pltpu.async_copy(est_hbm.at[pl.ds(base + c * 128, 128), :],
                         ebuf_v.at[c & 1], sem)

    fire(jnp.int32(0))

    def chunk(c, _):
        @pl.when(c < bpw // 128 - 1)
        def _():
            fire(c + 1)

        pltpu.make_async_copy(ost_hbm.at[pl.ds(0, 128), :],
                              obuf_v.at[c & 1], sem).wait()
        pltpu.make_async_copy(est_hbm.at[pl.ds(0, 128), :],
                              ebuf_v.at[c & 1], sem).wait()

        def group(j, _):
            dotv = jnp.zeros((L,), jnp.float32)
            pv = jnp.zeros((L,), jnp.float32)
            for r in range(L):
                row = j * L + r
                dot = jnp.zeros((L,), jnp.float32)
                on = jnp.zeros((L,), jnp.float32)
                en = jnp.zeros((L,), jnp.float32)
                for k in range(DIM // L):
                    o = obuf_v[c & 1, row, pl.ds(k * L, L)]
                    e = ebuf_v[c & 1, row, pl.ds(k * L, L)]
                    dot = dot + o * e
                    on = on + o * o
                    en = en + e * e
                sdot = jnp.sum(dot)
                sp = jnp.sum(on) * jnp.sum(en)
                dotv = jnp.where(lane == r, sdot, dotv)
                pv = jnp.where(lane == r, sp, pv)
            yi = jnp.int32(0x5F3759DF) - lax.shift_right_logical(
                plsc.bitcast(pv, jnp.int32), 1)
            y = plsc.bitcast(yi, jnp.float32)
            xh = pv * jnp.float32(0.5)
            for _ in range(3):
                y = y * (jnp.float32(1.5) - xh * y * y)
            out_v[pl.ds(c * 128 + j * L, L)] = dotv * y
            return 0

        lax.fori_loop(0, 128 // L, group, 0)
        return 0

    lax.fori_loop(0, bpw // 128, chunk, 0)
    pltpu.sync_copy(out_v, out_hbm.at[pl.ds(base, bpw)])


def kernel(o_id, e_id, mentors, mentees):
    batch = o_id.shape[0]
    mt = mentors.T
    me = mentees.T
    oid2 = o_id.reshape(batch // 128, 128)
    eid2 = e_id.reshape(batch // 128, 128)

    mesh = plsc.VectorSubcoreMesh(core_axis_name="c", subcore_axis_name="s",
                                  num_cores=NC, num_subcores=NS)
    params = pltpu.CompilerParams(needs_layout_passes=False,
                                  use_tc_tiling_on_sc=True)
    p1 = pl.kernel(
        _p1_body,
        out_type=(jax.ShapeDtypeStruct((STAG_ROWS, 2 * DIM), jnp.float32),
                  jax.ShapeDtypeStruct((STAG_ROWS, 2 * DIM), jnp.float32)),
        mesh=mesh,
        compiler_params=params,
        scratch_types=[
            pltpu.VMEM((batch // 128, 128), jnp.int32),
            pltpu.VMEM((batch + L,), jnp.int32),
            pltpu.VMEM((batch + L,), jnp.int32),
            pltpu.VMEM((NBUF, DIM, WINW), jnp.float32),
            pltpu.VMEM((DIM, SLIVER_W), jnp.float32),
            pltpu.VMEM((128, 2 * DIM), jnp.float32),
            pltpu.VMEM((128,), jnp.int32),
            pltpu.SMEM((4,), jnp.int32),
            pltpu.SemaphoreType.DMA,
            pltpu.SemaphoreType.DMA,
        ],
    )
    ost, est = p1(oid2, eid2, mt, me)

    p2 = pl.kernel(
        _p2_body,
        out_type=jax.ShapeDtypeStruct((batch,), jnp.float32),
        mesh=mesh,
        compiler_params=params,
        scratch_types=[
            pltpu.VMEM((2, 128, 2 * DIM), jnp.float32),
            pltpu.VMEM((2, 128, 2 * DIM), jnp.float32),
            pltpu.VMEM((batch // NW,), jnp.float32),
            pltpu.SemaphoreType.DMA,
        ],
    )
    return p2(ost, est)


# speedup vs baseline: 1.4610x; 1.4511x over previous
import functools

import jax
import jax.numpy as jnp
from jax import lax
from jax.experimental import pallas as pl
from jax.experimental.pallas import tpu as pltpu
from jax.experimental.pallas import tpu_sc as plsc

DIM = 64
L = 16
NC, NS = 2, 16
NW = NC * NS
BLK = 128
NV = 1000000
NBLK = -(-NV // BLK)
BPW_BLK = -(-NBLK // NW)
WINB = 2
WINW = WINB * BLK
NWIN = -(-BPW_BLK // WINB)
BMAX = NBLK - 3
SLIVER0 = (NBLK - 2) * BLK
SLIVER_W = NV - SLIVER0
BATCH = 16384
TRASH = BATCH
STAG_ROWS = BATCH + 8
NBUF = 4
POSB = 14
RING = 64
IDXCH = 32


def _dyn_lane(v, i):
    g = v.at[jnp.full((L,), i, jnp.int32)].get(mode="promise_in_bounds")
    return g[0]


def _p1_body(oid_hbm, eid_hbm, mt_hbm, me_hbm, ost_hbm, est_hbm,
             allidx_v, bkt_v, wl_v, chunk_v, sliver_v, ring_v, posring_v,
             cnt_s, sem_c, sem_s):
    wid = lax.axis_index("s") * NC + lax.axis_index("c")
    rlo = wid * (BPW_BLK * BLK)
    rhi = rlo + BPW_BLK * BLK
    lanei = lax.iota(jnp.int32, L)
    trash = jnp.full((L,), TRASH, jnp.int32)

    def one_pass(tbl, idxh, stag):
        for t in range(RING // L):
            posring_v[pl.ds(t * L, L)] = trash
        cnt_s[0] = jnp.int32(0)

        nb = jnp.int32(0)
        for q in range(BATCH // (IDXCH * 128)):
            pltpu.sync_copy(idxh.at[pl.ds(q * IDXCH, IDXCH)], allidx_v)

            def bk(v, nb, q=q):
                row = v >> 3
                seg = v & 7
                iv = allidx_v[row, pl.ds(seg * L, L)]
                m = (iv >= rlo) & (iv < rhi)
                pos = (q * IDXCH * 128 + v * L) + lanei
                packed = ((iv - rlo) << POSB) | pos
                plsc.store_compressed(bkt_v.at[pl.ds(nb, L)], packed, mask=m)
                return nb + plsc.all_reduce_population_count(m)[0]

            nb = lax.fori_loop(0, IDXCH * 128 // L, bk, nb)
        ngv = (nb + L - 1) >> 4
        sliver_loc = jnp.int32(SLIVER0) - rlo

        def extract_entry(ic_s, pos_s, buf):
            slot = cnt_s[0]
            colv = jnp.full((L,), ic_s, jnp.int32)
            for k in range(DIM // L):
                seg = plsc.load_gather(buf, [lanei + k * L, colv])
                ring_v[slot, pl.ds(k * L, L)] = seg
            base = (slot >> 4) * L
            pr = posring_v[pl.ds(base, L)]
            posring_v[pl.ds(base, L)] = jnp.where(lanei == (slot & (L - 1)),
                                                  pos_s, pr)
            nslot = slot + 1

            @pl.when(nslot == RING)
            def _():
                pltpu.async_copy(ring_v, stag.at[posring_v], sem_s).wait()
                for t in range(RING // L):
                    posring_v[pl.ds(t * L, L)] = trash

            cnt_s[0] = jnp.where(nslot == RING, 0, nslot)

        def process(buf, match, colbase):
            def pg(g, w):
                wv = bkt_v[pl.ds(g * L, L)]
                il = lax.shift_right_logical(wv, POSB)
                valid = (lanei + g * L) < nb
                m = match(il) & valid
                plsc.store_compressed(wl_v.at[pl.ds(w, L)], wv, mask=m)
                return w + plsc.all_reduce_population_count(m)[0]

            w = lax.fori_loop(0, ngv, pg, jnp.int32(0))

            def pe(e, _):
                grp = wl_v[pl.ds((e >> 4) * L, L)]
                sc = _dyn_lane(grp, e & (L - 1))
                il_s = lax.shift_right_logical(sc, POSB)
                pos_s = sc & ((1 << POSB) - 1)
                extract_entry(il_s - colbase, pos_s, buf)
                return 0

            lax.fori_loop(0, w, pe, 0)

        def fire(t):
            @pl.when(t < NWIN)
            def _():
                blk = jnp.minimum(wid * BPW_BLK + t * WINB, BMAX)
                start = pl.multiple_of(blk * BLK, BLK)
                for jb in range(DIM // 8):
                    pltpu.async_copy(
                        tbl.at[pl.ds(jb * 8, 8), pl.ds(start, WINW)],
                        chunk_v.at[t & (NBUF - 1), pl.ds(jb * 8, 8), :],
                        sem_c)

            @pl.when(t == NWIN)
            def _():
                for jb in range(DIM // 8):
                    pltpu.async_copy(
                        tbl.at[pl.ds(jb * 8, 8), pl.ds(SLIVER0, SLIVER_W)],
                        sliver_v.at[pl.ds(jb * 8, 8), :], sem_c)

        for t in range(NBUF - 1):
            fire(jnp.int32(t))

        def chunk_step(s, _):
            fire(s + (NBUF - 1))
            pltpu.make_async_copy(tbl.at[:, pl.ds(0, WINW)],
                                  chunk_v.at[s & (NBUF - 1)], sem_c).wait()
            process(chunk_v.at[s & (NBUF - 1)],
                    lambda il: ((il >> 8) == s) & (il < sliver_loc),
                    s * WINW)
            return 0

        lax.fori_loop(0, NWIN, chunk_step, 0)
        pltpu.make_async_copy(tbl.at[:, pl.ds(SLIVER0, SLIVER_W)],
                              sliver_v, sem_c).wait()
        process(sliver_v, lambda il: il >= sliver_loc, sliver_loc)
        pltpu.async_copy(ring_v, stag.at[posring_v], sem_s).wait()

    one_pass(mt_hbm, oid_hbm, ost_hbm)
    one_pass(me_hbm, eid_hbm, est_hbm)


def _p2_body(ost_hbm, est_hbm, out_hbm, obuf_v, ebuf_v, out_v, sem):
    wid = lax.axis_index("s") * NC + lax.axis_index("c")
    bpw = BATCH // NW
    base = wid * bpw
    lane = lax.iota(jnp.int32, L)

    def fire(c):
        pltpu.async_copy(ost_hbm.at[pl.ds(base + c * 128, 128), :],
                         obuf_v.at[c & 1], sem)
        pltpu.async_copy(est_hbm.at[pl.ds(base + c * 128, 128), :],
                         ebuf_v.at[c & 1], sem)

    fire(jnp.int32(0))

    def chunk(c, _):
        @pl.when(c < bpw // 128 - 1)
        def _():
            fire(c + 1)

        pltpu.make_async_copy(ost_hbm.at[pl.ds(0, 128), :],
                              obuf_v.at[c & 1], sem).wait()
        pltpu.make_async_copy(est_hbm.at[pl.ds(0, 128), :],
                              ebuf_v.at[c & 1], sem).wait()

        def group(j, _):
            dotv = jnp.zeros((L,), jnp.float32)
            pv = jnp.zeros((L,), jnp.float32)
            for r in range(L):
                row = j * L + r
                dot = jnp.zeros((L,), jnp.float32)
                on = jnp.zeros((L,), jnp.float32)
                en = jnp.zeros((L,), jnp.float32)
                for k in range(DIM // L):
                    o = obuf_v[c & 1, row, pl.ds(k * L, L)]
                    e = ebuf_v[c & 1, row, pl.ds(k * L, L)]
                    dot = dot + o * e
                    on = on + o * o
                    en = en + e * e
                sdot = jnp.sum(dot)
                sp = jnp.sum(on) * jnp.sum(en)
                dotv = jnp.where(lane == r, sdot, dotv)
                pv = jnp.where(lane == r, sp, pv)
            yi = jnp.int32(0x5F3759DF) - lax.shift_right_logical(
                plsc.bitcast(pv, jnp.int32), 1)
            y = plsc.bitcast(yi, jnp.float32)
            xh = pv * jnp.float32(0.5)
            for _ in range(3):
                y = y * (jnp.float32(1.5) - xh * y * y)
            out_v[pl.ds(c * 128 + j * L, L)] = dotv * y
            return 0

        lax.fori_loop(0, 128 // L, group, 0)
        return 0

    lax.fori_loop(0, bpw // 128, chunk, 0)
    pltpu.sync_copy(out_v, out_hbm.at[pl.ds(base, bpw)])


def kernel(o_id, e_id, mentors, mentees):
    batch = o_id.shape[0]
    mt = mentors.T
    me = mentees.T
    oid2 = o_id.reshape(batch // 128, 128)
    eid2 = e_id.reshape(batch // 128, 128)

    mesh = plsc.VectorSubcoreMesh(core_axis_name="c", subcore_axis_name="s",
                                  num_cores=NC, num_subcores=NS)
    params = pltpu.CompilerParams(needs_layout_passes=False,
                                  use_tc_tiling_on_sc=True)
    p1 = pl.kernel(
        _p1_body,
        out_type=(jax.ShapeDtypeStruct((STAG_ROWS, 2 * DIM), jnp.float32),
                  jax.ShapeDtypeStruct((STAG_ROWS, 2 * DIM), jnp.float32)),
        mesh=mesh,
        compiler_params=params,
        scratch_types=[
            pltpu.VMEM((IDXCH, 128), jnp.int32),
            pltpu.VMEM((batch + L,), jnp.int32),
            pltpu.VMEM((batch + L,), jnp.int32),
            pltpu.VMEM((NBUF, DIM, WINW), jnp.float32),
            pltpu.VMEM((DIM, SLIVER_W), jnp.float32),
            pltpu.VMEM((RING, 2 * DIM), jnp.float32),
            pltpu.VMEM((RING,), jnp.int32),
            pltpu.SMEM((4,), jnp.int32),
            pltpu.SemaphoreType.DMA,
            pltpu.SemaphoreType.DMA,
        ],
    )
    ost, est = p1(oid2, eid2, mt, me)

    p2 = pl.kernel(
        _p2_body,
        out_type=jax.ShapeDtypeStruct((batch,), jnp.float32),
        mesh=mesh,
        compiler_params=params,
        scratch_types=[
            pltpu.VMEM((2, 128, 2 * DIM), jnp.float32),
            pltpu.VMEM((2, 128, 2 * DIM), jnp.float32),
            pltpu.VMEM((batch // NW,), jnp.float32),
            pltpu.SemaphoreType.DMA,
        ],
    )
    return p2(ost, est)


# speedup vs baseline: 1.5149x; 1.0369x over previous
import functools

import jax
import jax.numpy as jnp
from jax import lax
from jax.experimental import pallas as pl
from jax.experimental.pallas import tpu as pltpu
from jax.experimental.pallas import tpu_sc as plsc

DIM = 64
L = 16
NC, NS = 2, 16
NW = NC * NS
BLK = 128
NV = 1000000
NBLK = -(-NV // BLK)
WINB = 2
WINW = WINB * BLK
NWIN = -(-NBLK // (NW * WINB))
BPW_BLK = NWIN * WINB
BMAX = NBLK - 3
SLIVER0 = (NBLK - 1) * BLK
SLIVER_W = NV - SLIVER0
BATCH = 16384
TRASH = BATCH
STAG_ROWS = BATCH + 8
NBUF = 5
SLSLOT = NWIN % NBUF
POSB = 14
RING = 48
IDXCH = 8


def _dyn_lane(v, i):
    g = v.at[jnp.full((L,), i, jnp.int32)].get(mode="promise_in_bounds")
    return g[0]


def _p1_body(oid_hbm, eid_hbm, mt_hbm, me_hbm, ost_hbm, est_hbm,
             allidx_v, bkt_v, wl_v, chunk_v, sliver_v, ring_v, posring_v,
             cnt_s, sem_c, sem_s):
    wid = lax.axis_index("s") * NC + lax.axis_index("c")
    rlo = wid * (BPW_BLK * BLK)
    rhi = rlo + BPW_BLK * BLK
    lanei = lax.iota(jnp.int32, L)
    trash = jnp.full((L,), TRASH, jnp.int32)

    def one_pass(tbl, idxh, stag):
        for t in range(RING // L):
            posring_v[pl.ds(t * L, L)] = trash
        cnt_s[0] = jnp.int32(0)

        nb = jnp.int32(0)
        for q in range(BATCH // (IDXCH * 128)):
            pltpu.sync_copy(idxh.at[pl.ds(q * IDXCH, IDXCH)], allidx_v)

            def bk(v, nb, q=q):
                row = v >> 3
                seg = v & 7
                iv = allidx_v[row, pl.ds(seg * L, L)]
                m = (iv >= rlo) & (iv < rhi)
                pos = (q * IDXCH * 128 + v * L) + lanei
                packed = ((iv - rlo) << POSB) | pos
                plsc.store_compressed(bkt_v.at[pl.ds(nb, L)], packed, mask=m)
                return nb + plsc.all_reduce_population_count(m)[0]

            nb = lax.fori_loop(0, IDXCH * 128 // L, bk, nb)
        ngv = (nb + L - 1) >> 4
        sliver_loc = jnp.int32(SLIVER0) - rlo

        def extract_entry(ic_s, pos_s, buf):
            slot = cnt_s[0]
            colv = jnp.full((L,), ic_s, jnp.int32)
            for k in range(DIM // L):
                seg = plsc.load_gather(buf, [lanei + k * L, colv])
                ring_v[slot, pl.ds(k * L, L)] = seg
            base = (slot >> 4) * L
            pr = posring_v[pl.ds(base, L)]
            posring_v[pl.ds(base, L)] = jnp.where(lanei == (slot & (L - 1)),
                                                  pos_s, pr)
            nslot = slot + 1

            @pl.when(nslot == RING)
            def _():
                pltpu.async_copy(ring_v, stag.at[posring_v], sem_s).wait()
                for t in range(RING // L):
                    posring_v[pl.ds(t * L, L)] = trash

            cnt_s[0] = jnp.where(nslot == RING, 0, nslot)

        def process(buf, match, colbase):
            def pg(g, w):
                wv = bkt_v[pl.ds(g * L, L)]
                il = lax.shift_right_logical(wv, POSB)
                valid = (lanei + g * L) < nb
                m = match(il) & valid
                plsc.store_compressed(wl_v.at[pl.ds(w, L)], wv, mask=m)
                return w + plsc.all_reduce_population_count(m)[0]

            w = lax.fori_loop(0, ngv, pg, jnp.int32(0))

            def pe(e, _):
                grp = wl_v[pl.ds((e >> 4) * L, L)]
                sc = _dyn_lane(grp, e & (L - 1))
                il_s = lax.shift_right_logical(sc, POSB)
                pos_s = sc & ((1 << POSB) - 1)
                extract_entry(il_s - colbase, pos_s, buf)
                return 0

            lax.fori_loop(0, w, pe, 0)

        def fire(t):
            slot = lax.rem(t, NBUF)

            @pl.when(t < NWIN)
            def _():
                blk = jnp.minimum(wid * BPW_BLK + t * WINB, BMAX)
                start = pl.multiple_of(blk * BLK, BLK)
                for jb in range(DIM // 8):
                    pltpu.async_copy(
                        tbl.at[pl.ds(jb * 8, 8), pl.ds(start, WINW)],
                        chunk_v.at[slot, pl.ds(jb * 8, 8), :],
                        sem_c)

            @pl.when(t == NWIN)
            def _():
                for jb in range(DIM // 8):
                    pltpu.async_copy(
                        tbl.at[pl.ds(jb * 8, 8), pl.ds(SLIVER0, SLIVER_W)],
                        sliver_v.at[pl.ds(jb * 8, 8), :], sem_c)

        for t in range(NBUF - 1):
            fire(jnp.int32(t))

        def chunk_step(s, _):
            fire(s + (NBUF - 1))
            slot = lax.rem(s, NBUF)
            pltpu.make_async_copy(tbl.at[:, pl.ds(0, WINW)],
                                  chunk_v.at[slot], sem_c).wait()
            process(chunk_v.at[slot],
                    lambda il: ((il >> 8) == s) & (il < sliver_loc),
                    s * WINW)
            return 0

        lax.fori_loop(0, NWIN, chunk_step, 0)
        pltpu.make_async_copy(tbl.at[:, pl.ds(SLIVER0, SLIVER_W)],
                              sliver_v, sem_c).wait()
        process(sliver_v, lambda il: il >= sliver_loc, sliver_loc)
        pltpu.async_copy(ring_v, stag.at[posring_v], sem_s).wait()

    one_pass(mt_hbm, oid_hbm, ost_hbm)
    one_pass(me_hbm, eid_hbm, est_hbm)


def _p2_body(ost_hbm, est_hbm, out_hbm, obuf_v, ebuf_v, out_v, sem):
    wid = lax.axis_index("s") * NC + lax.axis_index("c")
    bpw = BATCH // NW
    base = wid * bpw
    lane = lax.iota(jnp.int32, L)

    def fire(c):
        pltpu.async_copy(ost_hbm.at[pl.ds(base + c * 128, 128), :],
                         obuf_v.at[c & 1], sem)
        pltpu.async_copy(est_hbm.at[pl.ds(base + c * 128, 128), :],
                         ebuf_v.at[c & 1], sem)

    fire(jnp.int32(0))

    def chunk(c, _):
        @pl.when(c < bpw // 128 - 1)
        def _():
            fire(c + 1)

        pltpu.make_async_copy(ost_hbm.at[pl.ds(0, 128), :],
                              obuf_v.at[c & 1], sem).wait()
        pltpu.make_async_copy(est_hbm.at[pl.ds(0, 128), :],
                              ebuf_v.at[c & 1], sem).wait()

        def group(j, _):
            dotv = jnp.zeros((L,), jnp.float32)
            pv = jnp.zeros((L,), jnp.float32)
            for r in range(L):
                row = j * L + r
                dot = jnp.zeros((L,), jnp.float32)
                on = jnp.zeros((L,), jnp.float32)
                en = jnp.zeros((L,), jnp.float32)
                for k in range(DIM // L):
                    o = obuf_v[c & 1, row, pl.ds(k * L, L)]
                    e = ebuf_v[c & 1, row, pl.ds(k * L, L)]
                    dot = dot + o * e
                    on = on + o * o
                    en = en + e * e
                sdot = jnp.sum(dot)
                sp = jnp.sum(on) * jnp.sum(en)
                dotv = jnp.where(lane == r, sdot, dotv)
                pv = jnp.where(lane == r, sp, pv)
            yi = jnp.int32(0x5F3759DF) - lax.shift_right_logical(
                plsc.bitcast(pv, jnp.int32), 1)
            y = plsc.bitcast(yi, jnp.float32)
            xh = pv * jnp.float32(0.5)
            for _ in range(3):
                y = y * (jnp.float32(1.5) - xh * y * y)
            out_v[pl.ds(c * 128 + j * L, L)] = dotv * y
            return 0

        lax.fori_loop(0, 128 // L, group, 0)
        return 0

    lax.fori_loop(0, bpw // 128, chunk, 0)
    pltpu.sync_copy(out_v, out_hbm.at[pl.ds(base, bpw)])


def kernel(o_id, e_id, mentors, mentees):
    batch = o_id.shape[0]
    mt = mentors.T
    me = mentees.T
    oid2 = o_id.reshape(batch // 128, 128)
    eid2 = e_id.reshape(batch // 128, 128)

    mesh = plsc.VectorSubcoreMesh(core_axis_name="c", subcore_axis_name="s",
                                  num_cores=NC, num_subcores=NS)
    params = pltpu.CompilerParams(needs_layout_passes=False,
                                  use_tc_tiling_on_sc=True)
    p1 = pl.kernel(
        _p1_body,
        out_type=(jax.ShapeDtypeStruct((STAG_ROWS, 2 * DIM), jnp.float32),
                  jax.ShapeDtypeStruct((STAG_ROWS, 2 * DIM), jnp.float32)),
        mesh=mesh,
        compiler_params=params,
        scratch_types=[
            pltpu.VMEM((IDXCH, 128), jnp.int32),
            pltpu.VMEM((batch + L,), jnp.int32),
            pltpu.VMEM((batch + L,), jnp.int32),
            pltpu.VMEM((NBUF, DIM, WINW), jnp.float32),
            pltpu.VMEM((DIM, SLIVER_W), jnp.float32),
            pltpu.VMEM((RING, 2 * DIM), jnp.float32),
            pltpu.VMEM((RING,), jnp.int32),
            pltpu.SMEM((4,), jnp.int32),
            pltpu.SemaphoreType.DMA,
            pltpu.SemaphoreType.DMA,
        ],
    )
    ost, est = p1(oid2, eid2, mt, me)

    p2 = pl.kernel(
        _p2_body,
        out_type=jax.ShapeDtypeStruct((batch,), jnp.float32),
        mesh=mesh,
        compiler_params=params,
        scratch_types=[
            pltpu.VMEM((2, 128, 2 * DIM), jnp.float32),
            pltpu.VMEM((2, 128, 2 * DIM), jnp.float32),
            pltpu.VMEM((batch // NW,), jnp.float32),
            pltpu.SemaphoreType.DMA,
        ],
    )
    return p2(ost, est)
